# Initial kernel scaffold; baseline (speedup 1.0000x reference)
import jax
import jax.numpy as jnp
from jax.experimental import pallas as pl


def kernel(audio_codes, emb_tables, layer_weights, ln_gamma, ln_beta, W, b):
    raise NotImplementedError("write your pallas kernel here")


# speedup vs baseline: 1.9992x; 1.9992x over previous
import functools

import jax
import jax.numpy as jnp
from jax import lax
from jax.experimental import pallas as pl
from jax.experimental.pallas import tpu as pltpu
from jax.experimental.pallas import tpu_sc as plsc

B, L, S = 8, 8, 2048
K, D = 4096, 1024
TOK = B * S

NC = 2
NS = 16
NW = NC * NS
LANES = 16

TOK_PER_W = TOK // NW
C = 4
CR = C * L
NCHUNK = TOK_PER_W // C
DCHUNKS = D // LANES


def _sc_gather_sum(tables, idx, wbc):
    mesh = plsc.VectorSubcoreMesh(core_axis_name="c", subcore_axis_name="s")

    @functools.partial(
        pl.kernel,
        mesh=mesh,
        out_type=jax.ShapeDtypeStruct((TOK, D), jnp.float32),
        scratch_types=[
            pltpu.VMEM((TOK_PER_W * L,), jnp.int32),
            pltpu.VMEM((2, CR, D), jnp.float32),
            pltpu.VMEM((2, C, D), jnp.float32),
            pltpu.VMEM((L, LANES), jnp.float32),
            pltpu.SemaphoreType.DMA,
            pltpu.SemaphoreType.DMA,
            pltpu.SemaphoreType.DMA,
            pltpu.SemaphoreType.DMA,
        ],
    )
    def k(tab_hbm, idx_hbm, wb_hbm, out_hbm, idx_v, rows_v, acc_v, wv,
          sem_g0, sem_g1, sem_s0, sem_s1):
        wid = lax.axis_index("s") * NC + lax.axis_index("c")
        base = wid * TOK_PER_W

        pltpu.sync_copy(wb_hbm, wv)
        pltpu.sync_copy(idx_hbm.at[pl.ds(base * L, TOK_PER_W * L)], idx_v)

        sem_g = (sem_g0, sem_g1)
        sem_s = (sem_s0, sem_s1)

        def gather(c, b):
            return pltpu.make_async_copy(
                tab_hbm.at[idx_v.at[pl.ds(c * CR, CR)]], rows_v.at[b], sem_g[b])

        def store(c, b):
            return pltpu.make_async_copy(
                acc_v.at[b], out_hbm.at[pl.ds(base + c * C, C)], sem_s[b])

        gather(0, 0).start()
        gather(1, 1).start()

        ws = [wv[i, :] for i in range(L)]

        def accumulate(b):
            def dbody(dd, carry):
                off = dd * LANES
                for j in range(C):
                    s = None
                    for i in range(L):
                        r = rows_v[b, j * L + i, pl.ds(off, LANES)]
                        t = r * ws[i]
                        s = t if s is None else s + t
                    acc_v[b, j, pl.ds(off, LANES)] = s
                return carry
            lax.fori_loop(0, DCHUNKS, dbody, 0)

        def outer(c2, carry):
            for b in range(2):
                c = 2 * c2 + b
                gather(c, b).wait()

                @pl.when(c >= 2)
                def _():
                    store(c - 2, b).wait()

                accumulate(b)

                @pl.when(c + 2 < NCHUNK)
                def _():
                    gather(c + 2, b).start()

                store(c, b).start()
            return carry

        lax.fori_loop(0, NCHUNK // 2, outer, 0)
        store(NCHUNK - 2, 0).wait()
        store(NCHUNK - 1, 1).wait()

    return k(tables, idx, wbc)


def _tc_ln_proj(total, W, gamma, beta, bias):
    BM = 512

    def body(x_ref, w_ref, g_ref, bt_ref, bias_ref, o_ref):
        x = x_ref[...]
        mu = jnp.mean(x, axis=1, keepdims=True)
        xc = x - mu
        var = jnp.mean(xc * xc, axis=1, keepdims=True)
        xn = xc * lax.rsqrt(var + 1e-5)
        xn = xn * g_ref[...] + bt_ref[...]
        o_ref[...] = lax.dot_general(
            xn, w_ref[...], (((1,), (1,)), ((), ())),
            preferred_element_type=jnp.float32) + bias_ref[...]

    return pl.pallas_call(
        body,
        grid=(TOK // BM,),
        in_specs=[
            pl.BlockSpec((BM, D), lambda i: (i, 0)),
            pl.BlockSpec((D, D), lambda i: (0, 0)),
            pl.BlockSpec((1, D), lambda i: (0, 0)),
            pl.BlockSpec((1, D), lambda i: (0, 0)),
            pl.BlockSpec((1, D), lambda i: (0, 0)),
        ],
        out_specs=pl.BlockSpec((BM, D), lambda i: (i, 0)),
        out_shape=jax.ShapeDtypeStruct((TOK, D), jnp.float32),
    )(total, W, gamma, beta, bias)


def kernel(audio_codes, emb_tables, layer_weights, ln_gamma, ln_beta, W, b):
    codes = audio_codes.astype(jnp.int32)
    offs = (jnp.arange(L, dtype=jnp.int32) * K)[None, :, None]
    idx = jnp.transpose(codes + offs, (0, 2, 1)).reshape(TOK * L)
    tables = emb_tables.reshape(L * K, D)
    wbc = jnp.broadcast_to(layer_weights[:, None], (L, LANES))

    total = _sc_gather_sum(tables, idx, wbc)
    out = _tc_ln_proj(total, W, ln_gamma.reshape(1, D), ln_beta.reshape(1, D),
                      b.reshape(1, D))
    return out.reshape(B, S, D)
